# Initial kernel scaffold; baseline (speedup 1.0000x reference)
import jax
import jax.numpy as jnp
from jax.experimental import pallas as pl


def kernel(feats, edge_index, W, b, W_res, b_res, gamma, beta):
    raise NotImplementedError("write your pallas kernel here")


# speedup vs baseline: 4.4054x; 4.4054x over previous
import functools

import jax
import jax.numpy as jnp
from jax import lax
from jax.experimental import pallas as pl
from jax.experimental.pallas import tpu as pltpu
from jax.experimental.pallas import tpu_sc as plsc

N_NODES = 10000
D = 128
BN_EPS = 1e-5

NW = 32
N_PAD = 10240
ROWS_PER_TILE = N_PAD // 16
CHUNK = 128


def _make_sc_agg(e_pad):
    epw = e_pad // NW
    n_chunks = epw // CHUNK
    mesh = plsc.VectorSubcoreMesh(core_axis_name="c", subcore_axis_name="s")

    @functools.partial(
        pl.kernel,
        out_type=jax.ShapeDtypeStruct((2, N_PAD, D), jnp.float32),
        mesh=mesh,
        scratch_types=[
            pltpu.VMEM((CHUNK,), jnp.int32),
            pltpu.VMEM((CHUNK,), jnp.int32),
            pltpu.VMEM((CHUNK, D), jnp.float32),
            pltpu.VMEM_SHARED((N_PAD, D), jnp.float32),
            pltpu.SemaphoreType.DMA,
        ],
    )
    def sc_agg(feats_hbm, src_hbm, dst_hbm, out_hbm, src_v, dst_v, rows_v, acc_sh, sem):
        cid = lax.axis_index("c")
        sid = lax.axis_index("s")
        wid = sid * 2 + cid

        def zrow(i, _):
            for j in range(D // 16):
                rows_v[i, pl.ds(j * 16, 16)] = jnp.zeros((16,), jnp.float32)
            return 0

        lax.fori_loop(0, CHUNK, zrow, 0)
        for j in range(ROWS_PER_TILE // CHUNK):
            pltpu.sync_copy(
                rows_v, acc_sh.at[pl.ds(sid * ROWS_PER_TILE + j * CHUNK, CHUNK)]
            )
        plsc.subcore_barrier()

        base = wid * epw

        def body(c, _):
            off = base + c * CHUNK
            pltpu.sync_copy(src_hbm.at[pl.ds(off, CHUNK)], src_v)
            pltpu.sync_copy(dst_hbm.at[pl.ds(off, CHUNK)], dst_v)
            pltpu.async_copy(feats_hbm.at[src_v], rows_v, sem).wait()
            pltpu.sync_copy(rows_v, acc_sh.at[dst_v], add=True)
            return 0

        lax.fori_loop(0, n_chunks, body, 0)
        plsc.subcore_barrier()

        pltpu.sync_copy(
            acc_sh.at[pl.ds(sid * ROWS_PER_TILE, ROWS_PER_TILE)],
            out_hbm.at[cid, pl.ds(sid * ROWS_PER_TILE, ROWS_PER_TILE)],
        )

    return sc_agg


def _tc_dense_body(agg2_ref, feats_ref, w_ref, b_ref, wr_ref, br_ref, g_ref, bt_ref, out_ref):
    agg = (agg2_ref[0] + agg2_ref[1])[:N_NODES]
    h = jnp.maximum(
        jax.lax.dot(agg, w_ref[...], preferred_element_type=jnp.float32) + b_ref[...],
        0.0,
    )
    res = jnp.maximum(
        jax.lax.dot(feats_ref[...], wr_ref[...], preferred_element_type=jnp.float32)
        + br_ref[...],
        0.0,
    )
    h = h + res
    mean = jnp.mean(h, axis=0, keepdims=True)
    c = h - mean
    var = jnp.mean(c * c, axis=0, keepdims=True)
    out_ref[...] = c * jax.lax.rsqrt(var + BN_EPS) * g_ref[...] + bt_ref[...]


def kernel(feats, edge_index, W, b, W_res, b_res, gamma, beta):
    e = edge_index.shape[1]
    ei = edge_index.astype(jnp.int32)
    e_pad = -(-e // (NW * CHUNK)) * (NW * CHUNK)
    pad = e_pad - e
    src = jnp.concatenate([ei[0], jnp.zeros((pad,), jnp.int32)])
    dst = jnp.concatenate([ei[1], jnp.full((pad,), N_NODES, jnp.int32)])

    agg2 = _make_sc_agg(e_pad)(feats, src, dst)

    return pl.pallas_call(
        _tc_dense_body,
        out_shape=jax.ShapeDtypeStruct((N_NODES, D), jnp.float32),
    )(
        agg2,
        feats,
        W,
        b.reshape(1, D),
        W_res,
        b_res.reshape(1, D),
        gamma.reshape(1, D),
        beta.reshape(1, D),
    )
